# Initial kernel scaffold; baseline (speedup 1.0000x reference)
import jax
import jax.numpy as jnp
from jax.experimental import pallas as pl


def kernel(x, Wb0, bb0, Wb1, bb1, Wb2, bb2, Wcb, bcb, Wch, bch, Wlb, blb, Wlh, blh):
    raise NotImplementedError("write your pallas kernel here")


# speedup vs baseline: 2.2244x; 2.2244x over previous
import functools

import jax
import jax.numpy as jnp
import numpy as np
from jax.experimental import pallas as pl

B = 256
MAX_NODES = 1023
NPAD = 1024
N_FEAT = 32
HID = 64
LEAF_IDX = 0
FEAT_IDX = 2046
THR_IDX = 2078
OP_IDX = 2088
EOS_IDX = 4134
P_DIM = 4135

BT = 8


def _coef_tables():
    deg = np.zeros(NPAD, np.float64)
    deg[0] = 3.0
    deg[1:511] = 4.0
    deg[511:MAX_NODES] = 2.0
    a = np.zeros(NPAD, np.float64)
    p = np.zeros(NPAD, np.float64)
    c = np.zeros(NPAD, np.float64)
    a[:MAX_NODES] = 1.0 / deg[:MAX_NODES]
    d = np.arange(1, MAX_NODES)
    p[d] = 1.0 / np.sqrt(deg[(d - 1) // 2] * deg[d])
    d = np.arange(0, 511)
    c[d] = 1.0 / np.sqrt(deg[2 * d + 1] * deg[d])
    return np.stack([a, p, c], axis=1).astype(np.float32)


_COEF = _coef_tables()


def _gcn_agg(hw, a, p, c):
    bt, n, f = hw.shape
    zrow = jnp.zeros((bt, 1, f), hw.dtype)
    up = jnp.broadcast_to(hw[:, :512, None, :], (bt, 512, 2, f)).reshape(bt, n, f)
    par = jnp.concatenate([zrow, up[:, : n - 1, :]], axis=1)
    g = jnp.concatenate([hw[:, 1:, :], zrow], axis=1)
    s = g.reshape(bt, 512, 2, f).sum(axis=2)
    s = jnp.concatenate([s, jnp.zeros((bt, 512, f), hw.dtype)], axis=1)
    return a * hw + p * par + c * s


def _leaky(x):
    return jnp.where(x >= 0, x, 0.01 * x)


def _fwd_kernel(coef_ref, x_ref, wb0_ref, bb0_ref, wb1_ref, bb1_ref,
                wb2_ref, bb2_ref, wcb_ref, bcb_ref, wch_ref, bch_ref,
                wlb_ref, blb_ref, wlh_ref, blh_ref, out_ref):
    bt = x_ref.shape[0]
    a = coef_ref[:, 0:1][None]
    p = coef_ref[:, 1:2][None]
    c = coef_ref[:, 2:3][None]

    def gcn(h, w_ref, b_ref):
        f_out = w_ref.shape[1]
        hw = jnp.dot(h.reshape(bt * NPAD, h.shape[-1]), w_ref[...],
                     preferred_element_type=jnp.float32)
        hw = hw.reshape(bt, NPAD, f_out)
        return _gcn_agg(hw, a, p, c) + b_ref[...][None]

    x = x_ref[...]
    h = _leaky(gcn(x, wb0_ref, bb0_ref))
    h = _leaky(gcn(h, wb1_ref, bb1_ref))
    h = _leaky(gcn(h, wb2_ref, bb2_ref))
    hc = _leaky(gcn(h, wcb_ref, bcb_ref))
    out_c = gcn(hc, wch_ref, bch_ref).reshape(bt, 2 * NPAD)[:, : 2 * MAX_NODES]
    hl = _leaky(gcn(h, wlb_ref, blb_ref))
    out_l = gcn(hl, wlh_ref, blh_ref).reshape(bt, 2 * NPAD)[:, : 2 * MAX_NODES]

    stage = x[:, MAX_NODES, 0].reshape(bt, 1)
    m0 = (stage == 0.0).astype(jnp.float32)
    m1 = (stage == 1.0).astype(jnp.float32)
    m2 = (stage == 2.0).astype(jnp.float32)
    m3 = (stage == 3.0).astype(jnp.float32)

    out_ref[:, LEAF_IDX:FEAT_IDX] = m1 * out_l
    out_ref[:, FEAT_IDX:THR_IDX] = jnp.broadcast_to(m2, (bt, THR_IDX - FEAT_IDX))
    out_ref[:, THR_IDX:OP_IDX] = jnp.broadcast_to(m3, (bt, OP_IDX - THR_IDX))
    out_ref[:, OP_IDX:EOS_IDX] = m0 * out_c
    out_ref[:, EOS_IDX:P_DIM] = m0


@functools.partial(jax.jit, static_argnames=())
def kernel(x, Wb0, bb0, Wb1, bb1, Wb2, bb2, Wcb, bcb, Wch, bch, Wlb, blb, Wlh, blh):
    coef = jnp.asarray(_COEF)
    biases = [b.reshape(1, -1) for b in (bb0, bb1, bb2, bcb, bch, blb, blh)]
    bb0, bb1, bb2, bcb, bch, blb, blh = biases

    full = lambda arr: pl.BlockSpec(arr.shape, lambda i: (0,) * arr.ndim)
    in_specs = [
        full(coef),
        pl.BlockSpec((BT, NPAD, N_FEAT), lambda i: (i, 0, 0)),
        full(Wb0), full(bb0), full(Wb1), full(bb1), full(Wb2), full(bb2),
        full(Wcb), full(bcb), full(Wch), full(bch),
        full(Wlb), full(blb), full(Wlh), full(blh),
    ]
    out = pl.pallas_call(
        _fwd_kernel,
        grid=(B // BT,),
        in_specs=in_specs,
        out_specs=pl.BlockSpec((BT, P_DIM), lambda i: (i, 0)),
        out_shape=jax.ShapeDtypeStruct((B, P_DIM), jnp.float32),
    )(coef, x, Wb0, bb0, Wb1, bb1, Wb2, bb2, Wcb, bcb, Wch, bch, Wlb, blb, Wlh, blh)
    return out


# speedup vs baseline: 2.7329x; 1.2286x over previous
import functools

import jax
import jax.numpy as jnp
import numpy as np
from jax.experimental import pallas as pl

B = 256
MAX_NODES = 1023
NPAD = 1024
HALF = 512
N_FEAT = 32
HID = 64
LEAF_IDX = 0
FEAT_IDX = 2046
THR_IDX = 2078
OP_IDX = 2088
EOS_IDX = 4134
P_DIM = 4135

BT = 8


def _coef_tables():
    deg = np.zeros(NPAD, np.float64)
    deg[0] = 3.0
    deg[1:511] = 4.0
    deg[511:MAX_NODES] = 2.0
    a = np.zeros(NPAD, np.float64)
    a[:MAX_NODES] = 1.0 / deg[:MAX_NODES]
    e = np.zeros(NPAD, np.float64)
    k = np.arange(0, 511)
    e[k] = 1.0 / np.sqrt(deg[2 * k + 1] * deg[k])
    return np.stack([a, e], axis=1).astype(np.float32)


_COEF = _coef_tables()


def _leaky(x):
    return jnp.maximum(x, 0.01 * x)


def _gcn_agg(hw, a, e):
    bt, n, f = hw.shape
    zrow = jnp.zeros((bt, 1, f), hw.dtype)
    rep = hw[:, :HALF, :] * e
    rep2 = jnp.broadcast_to(rep[:, :, None, :], (bt, HALF, 2, f)).reshape(bt, n, f)
    par_top = jnp.concatenate([zrow, rep2[:, : HALF - 1, :]], axis=1)
    par_bot = rep2[:, HALF - 1 : n - 1, :]
    g = jnp.concatenate([hw[:, 1:, :], zrow], axis=1)
    s = g.reshape(bt, HALF, 2, f).sum(axis=2) * e
    top = a[:, :HALF] * hw[:, :HALF] + par_top + s
    bot = a[:, HALF:] * hw[:, HALF:] + par_bot
    return jnp.concatenate([top, bot], axis=1)


def _fwd_kernel(coef_ref, x_ref, wb0_ref, bb0_ref, wb1_ref, bb1_ref,
                wb2_ref, bb2_ref, wbody_ref, bbody_ref, whead_ref, bhead_ref,
                out_ref):
    bt = x_ref.shape[0]
    a = coef_ref[:, 0:1][None]
    e = coef_ref[:HALF, 1:2][None]

    def gcn(h, w_ref, b_ref):
        f_out = w_ref.shape[1]
        hw = jnp.dot(h.reshape(bt * NPAD, h.shape[-1]), w_ref[...],
                     preferred_element_type=jnp.float32)
        hw = hw.reshape(bt, NPAD, f_out)
        return _gcn_agg(hw, a, e) + b_ref[...][None]

    x = x_ref[...]
    h = _leaky(gcn(x, wb0_ref, bb0_ref))
    h = _leaky(gcn(h, wb1_ref, bb1_ref))
    h = _leaky(gcn(h, wb2_ref, bb2_ref))
    hb = _leaky(gcn(h, wbody_ref, bbody_ref))
    out4 = gcn(hb, whead_ref, bhead_ref)
    out_c = out4[:, :, 0:2].reshape(bt, 2 * NPAD)[:, : 2 * MAX_NODES]
    out_l = out4[:, :, 2:4].reshape(bt, 2 * NPAD)[:, : 2 * MAX_NODES]

    stage = x[:, MAX_NODES, 0].reshape(bt, 1)
    m0 = (stage == 0.0).astype(jnp.float32)
    m1 = (stage == 1.0).astype(jnp.float32)
    m2 = (stage == 2.0).astype(jnp.float32)
    m3 = (stage == 3.0).astype(jnp.float32)

    out_ref[:, LEAF_IDX:FEAT_IDX] = m1 * out_l
    out_ref[:, FEAT_IDX:THR_IDX] = jnp.broadcast_to(m2, (bt, THR_IDX - FEAT_IDX))
    out_ref[:, THR_IDX:OP_IDX] = jnp.broadcast_to(m3, (bt, OP_IDX - THR_IDX))
    out_ref[:, OP_IDX:EOS_IDX] = m0 * out_c
    out_ref[:, EOS_IDX:P_DIM] = m0


@functools.partial(jax.jit, static_argnames=())
def kernel(x, Wb0, bb0, Wb1, bb1, Wb2, bb2, Wcb, bcb, Wch, bch, Wlb, blb, Wlh, blh):
    coef = jnp.asarray(_COEF)
    Wbody = jnp.concatenate([Wcb, Wlb], axis=1)
    bbody = jnp.concatenate([bcb, blb]).reshape(1, -1)
    zz = jnp.zeros((HID, 2), jnp.float32)
    Whead = jnp.concatenate(
        [jnp.concatenate([Wch, zz], axis=1),
         jnp.concatenate([zz, Wlh], axis=1)], axis=0)
    bhead = jnp.concatenate([bch, blh]).reshape(1, -1)
    bb0, bb1, bb2 = (b.reshape(1, -1) for b in (bb0, bb1, bb2))

    full = lambda arr: pl.BlockSpec(arr.shape, lambda i: (0,) * arr.ndim)
    in_specs = [
        full(coef),
        pl.BlockSpec((BT, NPAD, N_FEAT), lambda i: (i, 0, 0)),
        full(Wb0), full(bb0), full(Wb1), full(bb1), full(Wb2), full(bb2),
        full(Wbody), full(bbody), full(Whead), full(bhead),
    ]
    out = pl.pallas_call(
        _fwd_kernel,
        grid=(B // BT,),
        in_specs=in_specs,
        out_specs=pl.BlockSpec((BT, P_DIM), lambda i: (i, 0)),
        out_shape=jax.ShapeDtypeStruct((B, P_DIM), jnp.float32),
    )(coef, x, Wb0, bb0, Wb1, bb1, Wb2, bb2, Wbody, bbody, Whead, bhead)
    return out


# speedup vs baseline: 2.9072x; 1.0638x over previous
import functools

import jax
import jax.numpy as jnp
import numpy as np
from jax.experimental import pallas as pl

B = 256
MAX_NODES = 1023
NPAD = 1024
HALF = 512
N_FEAT = 32
HID = 64
LEAF_IDX = 0
FEAT_IDX = 2046
THR_IDX = 2078
OP_IDX = 2088
EOS_IDX = 4134
P_DIM = 4135

BT = 16


def _coef_tables():
    deg = np.zeros(NPAD, np.float64)
    deg[0] = 3.0
    deg[1:511] = 4.0
    deg[511:MAX_NODES] = 2.0
    a = np.zeros(NPAD, np.float64)
    a[:MAX_NODES] = 1.0 / deg[:MAX_NODES]
    e = np.zeros(NPAD, np.float64)
    k = np.arange(0, 511)
    e[k] = 1.0 / np.sqrt(deg[2 * k + 1] * deg[k])
    return np.stack([a, e], axis=1).astype(np.float32)


_COEF = _coef_tables()


def _leaky(x):
    return jnp.maximum(x, 0.01 * x)


def _gcn_agg(hw, a, e):
    bt, n, f = hw.shape
    zrow = jnp.zeros((bt, 1, f), hw.dtype)
    rep = hw[:, :HALF, :] * e
    rep2 = jnp.broadcast_to(rep[:, :, None, :], (bt, HALF, 2, f)).reshape(bt, n, f)
    par_top = jnp.concatenate([zrow, rep2[:, : HALF - 1, :]], axis=1)
    par_bot = rep2[:, HALF - 1 : n - 1, :]
    g = jnp.concatenate([hw[:, 1:, :], zrow], axis=1)
    s = g.reshape(bt, HALF, 2, f).sum(axis=2) * e
    top = a[:, :HALF] * hw[:, :HALF] + par_top + s
    bot = a[:, HALF:] * hw[:, HALF:] + par_bot
    return jnp.concatenate([top, bot], axis=1)


def _fwd_kernel(coef_ref, x_ref, wb0_ref, bb0_ref, wb1_ref, bb1_ref,
                wb2_ref, bb2_ref, wbody_ref, bbody_ref, whead_ref, bhead_ref,
                out_ref):
    bt = x_ref.shape[0]
    a = coef_ref[:, 0:1][None]
    e = coef_ref[:HALF, 1:2][None]

    def gcn(h, w_ref, b_ref):
        f_out = w_ref.shape[1]
        hw = jnp.dot(h.reshape(bt * NPAD, h.shape[-1]), w_ref[...],
                     preferred_element_type=jnp.float32)
        hw = hw.reshape(bt, NPAD, f_out)
        return _gcn_agg(hw, a, e) + b_ref[...][None]

    x = x_ref[...]
    h = _leaky(gcn(x, wb0_ref, bb0_ref))
    h = _leaky(gcn(h, wb1_ref, bb1_ref))
    h = _leaky(gcn(h, wb2_ref, bb2_ref))
    hb = _leaky(gcn(h, wbody_ref, bbody_ref))
    out4 = gcn(hb, whead_ref, bhead_ref)
    out_c = out4[:, :, 0:2].reshape(bt, 2 * NPAD)[:, : 2 * MAX_NODES]
    out_l = out4[:, :, 2:4].reshape(bt, 2 * NPAD)[:, : 2 * MAX_NODES]

    stage = x[:, MAX_NODES, 0].reshape(bt, 1)
    m0 = (stage == 0.0).astype(jnp.float32)
    m1 = (stage == 1.0).astype(jnp.float32)
    m2 = (stage == 2.0).astype(jnp.float32)
    m3 = (stage == 3.0).astype(jnp.float32)

    out_ref[:, LEAF_IDX:FEAT_IDX] = m1 * out_l
    out_ref[:, FEAT_IDX:THR_IDX] = jnp.broadcast_to(m2, (bt, THR_IDX - FEAT_IDX))
    out_ref[:, THR_IDX:OP_IDX] = jnp.broadcast_to(m3, (bt, OP_IDX - THR_IDX))
    out_ref[:, OP_IDX:EOS_IDX] = m0 * out_c
    out_ref[:, EOS_IDX:P_DIM] = m0


@functools.partial(jax.jit, static_argnames=())
def kernel(x, Wb0, bb0, Wb1, bb1, Wb2, bb2, Wcb, bcb, Wch, bch, Wlb, blb, Wlh, blh):
    coef = jnp.asarray(_COEF)
    Wbody = jnp.concatenate([Wcb, Wlb], axis=1)
    bbody = jnp.concatenate([bcb, blb]).reshape(1, -1)
    zz = jnp.zeros((HID, 2), jnp.float32)
    Whead = jnp.concatenate(
        [jnp.concatenate([Wch, zz], axis=1),
         jnp.concatenate([zz, Wlh], axis=1)], axis=0)
    bhead = jnp.concatenate([bch, blh]).reshape(1, -1)
    bb0, bb1, bb2 = (b.reshape(1, -1) for b in (bb0, bb1, bb2))

    full = lambda arr: pl.BlockSpec(arr.shape, lambda i: (0,) * arr.ndim)
    in_specs = [
        full(coef),
        pl.BlockSpec((BT, NPAD, N_FEAT), lambda i: (i, 0, 0)),
        full(Wb0), full(bb0), full(Wb1), full(bb1), full(Wb2), full(bb2),
        full(Wbody), full(bbody), full(Whead), full(bhead),
    ]
    out = pl.pallas_call(
        _fwd_kernel,
        grid=(B // BT,),
        in_specs=in_specs,
        out_specs=pl.BlockSpec((BT, P_DIM), lambda i: (i, 0)),
        out_shape=jax.ShapeDtypeStruct((B, P_DIM), jnp.float32),
    )(coef, x, Wb0, bb0, Wb1, bb1, Wb2, bb2, Wbody, bbody, Whead, bhead)
    return out
